# baseline (speedup 1.0000x reference)
import functools

import jax
import jax.numpy as jnp
from jax import lax
from jax.experimental import pallas as pl
from jax.experimental.pallas import tpu as pltpu
from jax.experimental.pallas import tpu_sc as plsc

B = 16
T = 512
S = 256
F = 5
D = S * F
PRCCD = 3
LAST_LANE = (S - 1) * F + PRCCD

E = 64


def _orders_body(och_ref, orders_ref, gorders_ref):
    b = pl.program_id(0)
    row = och_ref[0]

    lane = lax.broadcasted_iota(jnp.int32, (T, D), 1)
    is_close = (lane % F) == PRCCD
    nz = is_close & (row != 0.0)
    big = jnp.int32(D + 1)
    first_idx = jnp.min(jnp.where(nz, lane, big), axis=1, keepdims=True)
    starts = jnp.sum(jnp.where(lane == first_idx, row, 0.0), axis=1, keepdims=True)
    last = jnp.sum(jnp.where(lane == LAST_LANE, row, 0.0), axis=1, keepdims=True)
    has = starts != 0.0
    perf = jnp.where(has, (last - starts) / jnp.where(has, starts, 1.0), 0.0)

    ii = lax.broadcasted_iota(jnp.int32, (T, T), 0)
    jj = lax.broadcasted_iota(jnp.int32, (T, T), 1)
    diag = ii == jj
    perf_row = jnp.sum(jnp.where(diag, perf, 0.0), axis=0, keepdims=True)

    beats = (perf_row > perf) | ((perf_row == perf) & (jj < ii))
    rank = jnp.sum(beats.astype(jnp.int32), axis=1, keepdims=True)

    ord_row = jnp.sum(jnp.where(rank == jj, ii, 0), axis=0, keepdims=True)
    orders_ref[0] = ord_row
    gorders_ref[0] = ord_row + b * T


def _compute_orders(och_flat):
    return pl.pallas_call(
        _orders_body,
        grid=(B,),
        in_specs=[pl.BlockSpec((1, T, D), lambda b: (b, 0, 0))],
        out_specs=[
            pl.BlockSpec((1, 1, T), lambda b: (b, 0, 0)),
            pl.BlockSpec((1, 1, T), lambda b: (b, 0, 0)),
        ],
        out_shape=[
            jax.ShapeDtypeStruct((B, 1, T), jnp.int32),
            jax.ShapeDtypeStruct((B, 1, T), jnp.int32),
        ],
    )(och_flat)


_NC = 2
_NS = 16
_NW = _NC * _NS
_ROWS = B * T
_RPW = _ROWS // _NW
_CH = 32
_NCHUNK = _RPW // _CH


def _gather_body(och_hbm, ab_hbm, idx_hbm,
                 out_och, out_ab,
                 idx_v, och_v, ab_v, sem):
    wid = lax.axis_index("s") * _NC + lax.axis_index("c")
    base = wid * _RPW
    for c in range(_NCHUNK):
        off = base + c * _CH
        pltpu.sync_copy(idx_hbm.at[pl.ds(off, _CH)], idx_v)
        h0 = pltpu.async_copy(och_hbm.at[idx_v], och_v, sem)
        h1 = pltpu.async_copy(ab_hbm.at[idx_v], ab_v, sem)
        h0.wait()
        h1.wait()
        pltpu.sync_copy(och_v, out_och.at[pl.ds(off, _CH)])
        pltpu.sync_copy(ab_v, out_ab.at[pl.ds(off, _CH)])


@functools.lru_cache(maxsize=1)
def _make_sc_gather():
    return pl.kernel(
        _gather_body,
        out_type=(
            jax.ShapeDtypeStruct((_ROWS, D), jnp.float32),
            jax.ShapeDtypeStruct((_ROWS, 2 * E), jnp.float32),
        ),
        mesh=plsc.VectorSubcoreMesh(core_axis_name="c", subcore_axis_name="s"),
        scratch_types=(
            pltpu.VMEM((_CH,), jnp.int32),
            pltpu.VMEM((_CH, D), jnp.float32),
            pltpu.VMEM((_CH, 2 * E), jnp.float32),
            pltpu.SemaphoreType.DMA,
        ),
    )


def kernel(in0, in1, ochlv):
    och_flat = ochlv.reshape(B, T, D)
    orders3, gorders3 = _compute_orders(och_flat)
    orders = orders3.reshape(B, T)
    gidx = gorders3.reshape(_ROWS)
    ab = jnp.concatenate([in0, in1], axis=-1).reshape(_ROWS, 2 * E)
    out_och, out_ab = _make_sc_gather()(
        och_flat.reshape(_ROWS, D),
        ab,
        gidx,
    )
    out0 = out_ab[:, :E].reshape(B, T, E)
    out1 = out_ab[:, E:].reshape(B, T, E)
    out2 = out_och.reshape(B, T, S, F)
    return ((out0, out1, out2), orders)


# speedup vs baseline: 1.5336x; 1.5336x over previous
import functools

import jax
import jax.numpy as jnp
from jax import lax
from jax.experimental import pallas as pl
from jax.experimental.pallas import tpu as pltpu
from jax.experimental.pallas import tpu_sc as plsc

B = 16
T = 512
S = 256
F = 5
PRCCD = 3
E = 64

_CHK = 32


def _orders_body(closes_ref, orders_ref):
    big = jnp.int32(S)

    def scan_chunk(c, carry):
        fidx, fval = carry
        cl = closes_ref[pl.ds(c * _CHK, _CHK)]
        tv = lax.broadcasted_iota(jnp.int32, (_CHK, B, T), 0) + c * _CHK
        nz = cl != 0.0
        cf = jnp.min(jnp.where(nz, tv, big), axis=0)
        cv = jnp.sum(jnp.where(tv == cf[None], cl, 0.0), axis=0)
        take = cf < fidx
        return (jnp.where(take, cf, fidx), jnp.where(take, cv, fval))

    init = (jnp.full((B, T), big, jnp.int32), jnp.zeros((B, T), jnp.float32))
    _, starts = lax.fori_loop(0, S // _CHK, scan_chunk, init)
    last = closes_ref[S - 1]
    has = starts != 0.0
    perf = jnp.where(has, (last - starts) / jnp.where(has, starts, 1.0), 0.0)

    ii = lax.broadcasted_iota(jnp.int32, (T, T), 0)
    jj = lax.broadcasted_iota(jnp.int32, (T, T), 1)
    diag = ii == jj
    for b in range(B):
        pr = perf[b:b + 1, :]
        pc = jnp.sum(jnp.where(diag, pr, 0.0), axis=1, keepdims=True)
        beats = (pr > pc) | ((pr == pc) & (jj < ii))
        rank = jnp.sum(beats.astype(jnp.int32), axis=1, keepdims=True)
        row = jnp.sum(jnp.where(rank == jj, ii, 0), axis=0, keepdims=True)
        orders_ref[b:b + 1, :] = row


def _compute_orders(closes):
    return pl.pallas_call(
        _orders_body,
        grid=(1,),
        in_specs=[pl.BlockSpec((S, B, T), lambda i: (0, 0, 0))],
        out_specs=pl.BlockSpec((B, T), lambda i: (0, 0)),
        out_shape=jax.ShapeDtypeStruct((B, T), jnp.int32),
    )(closes)


_NC = 2
_NS = 16
_NW = _NC * _NS
_PLANES = S * F
_PPW = _PLANES // _NW
_KU = 4


def _permute_tile(src_v, dst_v, ord_v, nrows, obase_of_row):
    iota16 = lax.iota(jnp.int32, 16)
    nchunk = T // 16

    @plsc.parallel_loop(0, nrows * nchunk, unroll=8)
    def _(g):
        r = lax.shift_right_logical(g, 5)
        k = (g & (nchunk - 1)) * 16
        rvec = jnp.full((16,), r, jnp.int32)
        idx = ord_v[pl.ds(obase_of_row(r) + k, 16)]
        vals = plsc.load_gather(src_v, [rvec, idx])
        plsc.store_scatter(dst_v, [rvec, k + iota16], vals)


def _gather_body(x_hbm, a0_hbm, a1_hbm, ord_hbm,
                 y_hbm, b0_hbm, b1_hbm,
                 ord_v, xin_v, xout_v, ain_v, aout_v):
    wid = lax.axis_index("s") * _NC + lax.axis_index("c")
    pltpu.sync_copy(ord_hbm, ord_v)

    p0 = wid * _PPW

    def plane_body(i, _):
        p = p0 + i
        pltpu.sync_copy(x_hbm.at[p], xin_v)
        _permute_tile(xin_v, xout_v, ord_v, B, lambda r: r * T)
        pltpu.sync_copy(xout_v, y_hbm.at[p])
        return 0

    lax.fori_loop(0, _PPW, plane_body, 0)

    bb = wid // 2

    def do_a(src_hbm, dst_hbm):
        pltpu.sync_copy(src_hbm.at[bb], ain_v)
        _permute_tile(ain_v, aout_v, ord_v, E, lambda r: bb * T)
        pltpu.sync_copy(aout_v, dst_hbm.at[bb])

    @pl.when(wid % 2 == 0)
    def _():
        do_a(a0_hbm, b0_hbm)

    @pl.when(wid % 2 == 1)
    def _():
        do_a(a1_hbm, b1_hbm)


@functools.lru_cache(maxsize=1)
def _make_sc_gather():
    return pl.kernel(
        _gather_body,
        out_type=(
            jax.ShapeDtypeStruct((_PLANES, B, T), jnp.float32),
            jax.ShapeDtypeStruct((B, E, T), jnp.float32),
            jax.ShapeDtypeStruct((B, E, T), jnp.float32),
        ),
        mesh=plsc.VectorSubcoreMesh(core_axis_name="c", subcore_axis_name="s"),
        scratch_types=(
            pltpu.VMEM((B * T,), jnp.int32),
            pltpu.VMEM((B, T), jnp.float32),
            pltpu.VMEM((B, T), jnp.float32),
            pltpu.VMEM((E, T), jnp.float32),
            pltpu.VMEM((E, T), jnp.float32),
        ),
        compiler_params=pltpu.CompilerParams(needs_layout_passes=False),
    )


def kernel(in0, in1, ochlv):
    xt = jnp.transpose(ochlv, (2, 3, 0, 1))
    closes = xt[:, PRCCD]
    orders = _compute_orders(closes)
    ord_flat = orders.reshape(B * T)

    x = xt.reshape(_PLANES, B, T)
    a0 = jnp.transpose(in0, (0, 2, 1))
    a1 = jnp.transpose(in1, (0, 2, 1))
    y, b0, b1 = _make_sc_gather()(x, a0, a1, ord_flat)

    out2 = jnp.transpose(y.reshape(S, F, B, T), (2, 3, 0, 1))
    out0 = jnp.transpose(b0, (0, 2, 1))
    out1 = jnp.transpose(b1, (0, 2, 1))
    return ((out0, out1, out2), orders)
